# Initial kernel scaffold; baseline (speedup 1.0000x reference)
import jax
import jax.numpy as jnp
from jax.experimental import pallas as pl


def kernel(x, edge_index, W1, b1, W2, b2, Wm1, bm1, gamma, beta, Wm2, bm2):
    raise NotImplementedError("write your pallas kernel here")


# speedup vs baseline: 4.4258x; 4.4258x over previous
import functools

import jax
import jax.numpy as jnp
from jax import lax
from jax.experimental import pallas as pl
from jax.experimental.pallas import tpu as pltpu
from jax.experimental.pallas import tpu_sc as plsc

N = 10000
E = 320000
D = 128
MH = 200
MHP = 256
NPAD = 10240
BM = 1024
NC = 2
NS = 16
EB = 128
NTILES = NC * NS
CB = 8
NB = 80
EPAD = NTILES * NB * EB
RPT = NPAD // NS

_f32 = jnp.float32


def _sc_agg():
    mesh = plsc.VectorSubcoreMesh(core_axis_name="c", subcore_axis_name="s")
    scratch = [
        pltpu.VMEM_SHARED((NPAD, D), _f32),
        pltpu.VMEM((CB, EB), jnp.int32),
        pltpu.VMEM((CB, EB), jnp.int32),
        pltpu.VMEM((EB, D), _f32),
    ]

    def body(src_hbm, dst_hbm, y_hbm, z2_hbm, part_hbm, acc, idx_s, idx_d,
             rows):
        c = lax.axis_index("c")
        s = lax.axis_index("s")
        tile = c * NS + s
        pltpu.sync_copy(z2_hbm, acc.at[pl.ds(s * RPT, RPT)])
        plsc.subcore_barrier()

        def chunk(chi, carry):
            off = pl.multiple_of(chi * CB, CB)
            pltpu.sync_copy(src_hbm.at[tile, pl.ds(off, CB)], idx_s)
            pltpu.sync_copy(dst_hbm.at[tile, pl.ds(off, CB)], idx_d)

            def step(b, cc):
                pltpu.sync_copy(y_hbm.at[idx_s.at[b]], rows)
                pltpu.sync_copy(rows, acc.at[idx_d.at[b]], add=True)
                return cc

            lax.fori_loop(0, CB, step, 0)
            return carry

        lax.fori_loop(0, NB // CB, chunk, 0)
        plsc.subcore_barrier()
        pltpu.sync_copy(acc.at[pl.ds(s * RPT, RPT)],
                        part_hbm.at[pl.ds(c * NPAD + s * RPT, RPT)])

    return pl.kernel(
        body, out_type=(jax.ShapeDtypeStruct((NC * NPAD, D), _f32),),
        mesh=mesh, scratch_types=scratch)


def _sc_cnt():
    mesh = plsc.VectorSubcoreMesh(core_axis_name="c", subcore_axis_name="s")
    scratch = [
        pltpu.VMEM_SHARED((NPAD, D), _f32),
        pltpu.VMEM((CB, EB), jnp.int32),
        pltpu.VMEM((EB, D), _f32),
    ]

    def body(dst_hbm, ones_hbm, z2_hbm, cnt_hbm, acc, idx_d, ones_v):
        c = lax.axis_index("c")
        s = lax.axis_index("s")
        tile = c * NS + s
        pltpu.sync_copy(z2_hbm, acc.at[pl.ds(s * RPT, RPT)])
        pltpu.sync_copy(ones_hbm, ones_v)
        plsc.subcore_barrier()

        def chunk(chi, carry):
            off = pl.multiple_of(chi * CB, CB)
            pltpu.sync_copy(dst_hbm.at[tile, pl.ds(off, CB)], idx_d)

            def step(b, cc):
                pltpu.sync_copy(ones_v, acc.at[idx_d.at[b]], add=True)
                return cc

            lax.fori_loop(0, CB, step, 0)
            return carry

        lax.fori_loop(0, NB // CB, chunk, 0)
        plsc.subcore_barrier()
        pltpu.sync_copy(acc.at[pl.ds(s * RPT, RPT)],
                        cnt_hbm.at[pl.ds(c * NPAD + s * RPT, RPT)])

    return pl.kernel(
        body, out_type=(jax.ShapeDtypeStruct((NC * NPAD, D), _f32),),
        mesh=mesh, scratch_types=scratch)


_DOT = functools.partial(jnp.dot, preferred_element_type=_f32,
                         precision=lax.Precision.HIGHEST)


def _mm_body(x_ref, w_ref, o_ref):
    o_ref[...] = _DOT(x_ref[...], w_ref[...])


def _layer_body(p0_ref, p1_ref, y_ref, c0_ref, c1_ref, b_ref, w_ref, o_ref):
    cnt = c0_ref[...] + c1_ref[...]
    recip = 1.0 / (cnt + 2.0)
    agg = p0_ref[...] + p1_ref[...] + 2.0 * y_ref[...]
    h = jnp.maximum(agg * recip + b_ref[...], 0.0)
    o_ref[...] = _DOT(h, w_ref[...])


def _mlp_body(p0_ref, p1_ref, y_ref, c0_ref, c1_ref, b_ref, wm_ref, bm_ref,
              z_ref, sum_ref, sq_ref):
    i = pl.program_id(0)
    cnt = c0_ref[...] + c1_ref[...]
    recip = 1.0 / (cnt + 2.0)
    agg = p0_ref[...] + p1_ref[...] + 2.0 * y_ref[...]
    h = jnp.maximum(agg * recip + b_ref[...], 0.0)
    z = jnp.maximum(_DOT(h, wm_ref[...]) + bm_ref[...], 0.0)
    z_ref[...] = z
    row = i * BM + lax.broadcasted_iota(jnp.int32, (BM, 1), 0)
    zm = jnp.where(row < N, z, 0.0)
    @pl.when(i == 0)
    def _():
        sum_ref[...] = jnp.zeros_like(sum_ref)
        sq_ref[...] = jnp.zeros_like(sq_ref)
    sum_ref[...] += jnp.sum(zm, axis=0, keepdims=True)
    sq_ref[...] += jnp.sum(zm * zm, axis=0, keepdims=True)


def _out_body(z_ref, sum_ref, sq_ref, g_ref, bt_ref, wm2_ref, bm2_ref, o_ref):
    inv_n = 1.0 / N
    mu = sum_ref[...] * inv_n
    var = sq_ref[...] * inv_n - mu * mu
    a = g_ref[...] * lax.rsqrt(var + 1e-5)
    zn = z_ref[...] * a + (bt_ref[...] - mu * a)
    v = _DOT(zn, wm2_ref[...]) + bm2_ref[...]
    o_ref[...] = 1.0 / (1.0 + jnp.exp(-v))


def _row_block(i):
    return (i, 0)


def _const_block(i):
    return (0, 0)


def _tc_matmul(x, w):
    return pl.pallas_call(
        _mm_body,
        grid=(NPAD // BM,),
        in_specs=[pl.BlockSpec((BM, x.shape[1]), _row_block),
                  pl.BlockSpec(w.shape, _const_block)],
        out_specs=pl.BlockSpec((BM, w.shape[1]), _row_block),
        out_shape=jax.ShapeDtypeStruct((NPAD, w.shape[1]), _f32),
    )(x, w)


def _tc_layer(p0, p1, y, c0, c1, brow, w):
    return pl.pallas_call(
        _layer_body,
        grid=(NPAD // BM,),
        in_specs=[pl.BlockSpec((BM, D), _row_block),
                  pl.BlockSpec((BM, D), _row_block),
                  pl.BlockSpec((BM, D), _row_block),
                  pl.BlockSpec((BM, 1), _row_block),
                  pl.BlockSpec((BM, 1), _row_block),
                  pl.BlockSpec((1, D), _const_block),
                  pl.BlockSpec((D, D), _const_block)],
        out_specs=pl.BlockSpec((BM, D), _row_block),
        out_shape=jax.ShapeDtypeStruct((NPAD, D), _f32),
    )(p0, p1, y, c0, c1, brow, w)


def _tc_mlp(p0, p1, y, c0, c1, brow, wm, bmrow):
    return pl.pallas_call(
        _mlp_body,
        grid=(NPAD // BM,),
        in_specs=[pl.BlockSpec((BM, D), _row_block),
                  pl.BlockSpec((BM, D), _row_block),
                  pl.BlockSpec((BM, D), _row_block),
                  pl.BlockSpec((BM, 1), _row_block),
                  pl.BlockSpec((BM, 1), _row_block),
                  pl.BlockSpec((1, D), _const_block),
                  pl.BlockSpec((D, MHP), _const_block),
                  pl.BlockSpec((1, MHP), _const_block)],
        out_specs=[pl.BlockSpec((BM, MHP), _row_block),
                   pl.BlockSpec((1, MHP), _const_block),
                   pl.BlockSpec((1, MHP), _const_block)],
        out_shape=[jax.ShapeDtypeStruct((NPAD, MHP), _f32),
                   jax.ShapeDtypeStruct((1, MHP), _f32),
                   jax.ShapeDtypeStruct((1, MHP), _f32)],
    )(p0, p1, y, c0, c1, brow, wm, bmrow)


def _tc_out(z, sums, sqs, grow, btrow, wm2, bm2row):
    return pl.pallas_call(
        _out_body,
        grid=(NPAD // BM,),
        in_specs=[pl.BlockSpec((BM, MHP), _row_block),
                  pl.BlockSpec((1, MHP), _const_block),
                  pl.BlockSpec((1, MHP), _const_block),
                  pl.BlockSpec((1, MHP), _const_block),
                  pl.BlockSpec((1, MHP), _const_block),
                  pl.BlockSpec((MHP, D), _const_block),
                  pl.BlockSpec((1, D), _const_block)],
        out_specs=pl.BlockSpec((BM, D), _row_block),
        out_shape=jax.ShapeDtypeStruct((NPAD, D), _f32),
    )(z, sums, sqs, grow, btrow, wm2, bm2row)


def kernel(x, edge_index, W1, b1, W2, b2, Wm1, bm1, gamma, beta, Wm2, bm2):
    ei = edge_index.astype(jnp.int32)
    pad_e = jnp.full((EPAD - E,), N, jnp.int32)
    src2d = jnp.concatenate([ei[0], pad_e]).reshape(NTILES, NB, EB)
    dst2d = jnp.concatenate([ei[1], pad_e]).reshape(NTILES, NB, EB)
    xp = jnp.zeros((NPAD, D), _f32).at[:N].set(x)
    z2 = jnp.zeros((RPT, D), _f32)
    ones2 = jnp.ones((EB, D), _f32)
    b1r = b1.reshape(1, D)
    b2r = b2.reshape(1, D)
    wm1p = jnp.zeros((D, MHP), _f32).at[:, :MH].set(Wm1)
    bm1r = jnp.zeros((1, MHP), _f32).at[0, :MH].set(bm1)
    gr = jnp.zeros((1, MHP), _f32).at[0, :MH].set(gamma)
    btr = jnp.zeros((1, MHP), _f32).at[0, :MH].set(beta)
    wm2p = jnp.zeros((MHP, D), _f32).at[:MH, 0:1].set(Wm2)
    bm2r = jnp.zeros((1, D), _f32) + bm2[0]

    agg = _sc_agg()
    cntk = _sc_cnt()

    (cnt,) = cntk(dst2d, ones2, z2)
    c0, c1 = cnt[:NPAD, 0:1], cnt[NPAD:, 0:1]
    y1 = _tc_matmul(xp, W1)
    (part,) = agg(src2d, dst2d, y1, z2)
    y2 = _tc_layer(part[:NPAD], part[NPAD:], y1, c0, c1, b1r, W2)
    (part2,) = agg(src2d, dst2d, y2, z2)
    z, sums, sqs = _tc_mlp(part2[:NPAD], part2[NPAD:], y2, c0, c1, b2r,
                           wm1p, bm1r)
    out = _tc_out(z, sums, sqs, gr, btr, wm2p, bm2r)
    return out[:N, 0:1]
